# Initial kernel scaffold; baseline (speedup 1.0000x reference)
import jax
import jax.numpy as jnp
from jax.experimental import pallas as pl


def kernel(logits, u):
    raise NotImplementedError("write your pallas kernel here")


# speedup vs baseline: 12.3308x; 12.3308x over previous
import functools
import jax
import jax.numpy as jnp
from jax import lax
from jax.experimental import pallas as pl

_ROWS = 128
_VOCAB = 100000
_K = 64
_RB = 8


def _body(x_ref, u_ref, samp_ref, conf_ref):
    x = x_ref[...]
    bits = lax.bitcast_convert_type(x, jnp.int32)
    key = jnp.where(bits < 0, bits ^ jnp.int32(0x7FFFFFFF), bits)

    kf = jnp.float32(_K)

    def count_ge(t):
        return jnp.sum((key >= t).astype(jnp.float32), axis=1, keepdims=True)

    neg = jnp.full((_RB, 1), jnp.int32(-2147483648))
    zero = jnp.zeros((_RB, 1), jnp.int32)
    base = jnp.where(count_ge(zero) >= kf, zero, neg)
    for b in range(30, -1, -1):
        cand = base | jnp.int32(1 << b)
        base = jnp.where(count_ge(cand) >= kf, cand, base)
    kth = base

    mask = key >= kth

    g = -jnp.log(-jnp.log(u_ref[...]))
    neg_inf = jnp.float32(-3.4e38)
    score = jnp.where(mask, x + g, neg_inf)
    smax = jnp.max(score, axis=1, keepdims=True)
    iota = lax.broadcasted_iota(jnp.int32, (_RB, _VOCAB), 1)
    big = jnp.int32(2**30)
    samp = jnp.min(jnp.where(score == smax, iota, big), axis=1, keepdims=True)

    m = jnp.max(x, axis=1, keepdims=True)
    e = jnp.where(mask, jnp.exp(x - m), jnp.float32(0.0))
    denom = jnp.sum(e, axis=1, keepdims=True)
    xs = jnp.sum(jnp.where(iota == samp, x, jnp.float32(0.0)), axis=1,
                 keepdims=True)
    conf = jnp.exp(xs - m) / denom

    samp_ref[...] = samp
    conf_ref[...] = conf


@jax.jit
def kernel(logits, u):
    grid = _ROWS // _RB
    samp, conf = pl.pallas_call(
        _body,
        grid=(grid,),
        in_specs=[
            pl.BlockSpec((_RB, _VOCAB), lambda i: (i, 0)),
            pl.BlockSpec((_RB, _VOCAB), lambda i: (i, 0)),
        ],
        out_specs=[
            pl.BlockSpec((_RB, 1), lambda i: (i, 0)),
            pl.BlockSpec((_RB, 1), lambda i: (i, 0)),
        ],
        out_shape=[
            jax.ShapeDtypeStruct((_ROWS, 1), jnp.int32),
            jax.ShapeDtypeStruct((_ROWS, 1), jnp.float32),
        ],
    )(logits, u)
    return samp[:, 0], conf[:, 0]
